# baseline (speedup 1.0000x reference)
import functools

import jax
import jax.numpy as jnp
import numpy as np
from jax import lax
from jax.experimental import pallas as pl
from jax.experimental.pallas import tpu as pltpu
from jax.experimental.pallas import tpu_sc as plsc

WS = 9
WT = 1
PS = 7
K = 10
STRIDE0 = 4
T, C, H, W = 5, 3, 128, 128
ROWS = H + PS - 1
RSTR = 136
PLANE = ROWS * RSTR
FRAME = C * PLANE
FRAME_PAD = 54680
NH = H // STRIDE0
NWIN = WS * WS
NWIN_PAD = 96
NDT = 2 * WT + 1
NCHUNK = NWIN_PAD // 16
NSLOT = NDT * NWIN_PAD
NVREG = NSLOT // 16
NTILES = 32
NELEM = C * PS * PS
GSLICE = 17408

_INF = float(np.inf)
_MAGIC = float(1.5 * 2.0**23)

_OFFS = np.zeros((160,), np.int32)
_OFFS[:NELEM] = np.add.outer(
    np.arange(C) * PLANE,
    np.add.outer(np.arange(PS) * RSTR, np.arange(PS)).reshape(-1),
).reshape(-1)


def _sc_body(vp_hbm, flw_hbm, off_hbm, out_hbm,
             frame_a, frame_b, cb_v, flow_v, off_v, dist, stage,
             sem_a, sem_b):
    tid = lax.axis_index("s") * 2 + lax.axis_index("c")
    lane = lax.broadcasted_iota(jnp.int32, (16,), 0)
    qh_f = (tid * STRIDE0).astype(jnp.float32)
    pltpu.sync_copy(off_hbm, off_v)
    pltpu.sync_copy(flw_hbm.at[:, STRIDE0 * tid], flow_v)
    pltpu.sync_copy(vp_hbm.at[0], frame_a)
    pltpu.sync_copy(vp_hbm.at[0], frame_b)

    def t_body(t, total):
        def emit_cb(dtidx):
            frow = (t * NDT + dtidx) * 2
            zero16 = jnp.zeros((16,), jnp.int32)
            for g in range(2):
                qx = lane * STRIDE0 + 16 * STRIDE0 * g
                fw = plsc.load_gather(flow_v, [zero16 + frow, qx])
                fh = plsc.load_gather(flow_v, [zero16 + (frow + 1), qx])
                rw = (qx.astype(jnp.float32) + fw + _MAGIC) - _MAGIC
                rh = (qh_f + fh + _MAGIC) - _MAGIC
                qi = (lane + 16 * g) * NWIN_PAD

                @plsc.parallel_loop(0, NWIN)
                def w_body(w, rw=rw, rh=rh, qi=qi):
                    wi = (w // WS - WS // 2).astype(jnp.float32)
                    wj = (w % WS - WS // 2).astype(jnp.float32)
                    ch = jnp.minimum(jnp.maximum(rh + wi, 0.0), float(H - 1))
                    cw = jnp.minimum(jnp.maximum(rw + wj, 0.0), float(W - 1))
                    base = (ch * float(RSTR) + cw).astype(jnp.int32)
                    plsc.store_scatter(cb_v, [qi + w], base)

        def emit_pass(gref, dtidx):
            offv = [off_v[pl.ds(m * 16, 16)] for m in range(10)]

            @plsc.parallel_loop(0, NH, unroll=2)
            def q_body(q):
                qbase = tid * (STRIDE0 * RSTR) + q * STRIDE0
                bases = [cb_v[pl.ds(q * NWIN_PAD + k * 16, 16)]
                         for k in range(5)]
                accs = [jnp.zeros((16,), jnp.float32) for _ in range(5)]
                for c in range(C):
                    for i in range(PS):
                        off_ci = c * PLANE + i * RSTR
                        qrow = frame_a[pl.ds(qbase + off_ci, 16)]
                        for j in range(PS):
                            qv = qrow[j]
                            for k in range(5):
                                cv = plsc.load_gather(
                                    gref, [bases[k] + (off_ci + j)])
                                dd = cv - qv
                                accs[k] = accs[k] + dd * dd
                for k in range(5):
                    dist[q, pl.ds(dtidx * NWIN_PAD + k * 16, 16)] = accs[k]
                base80 = cb_v[pl.ds(q * NWIN_PAD + 80, 16)][0]
                acc80 = jnp.zeros((16,), jnp.float32)
                for m in range(10):
                    qe = plsc.load_gather(frame_a, [offv[m] + qbase])
                    ce = plsc.load_gather(gref, [offv[m] + base80])
                    dd = qe - ce
                    sq = dd * dd
                    if (m + 1) * 16 > NELEM:
                        sq = jnp.where(lane < NELEM - m * 16, sq,
                                       jnp.float32(0.0))
                    acc80 = acc80 + sq
                d80 = jnp.sum(acc80)
                dist[q, pl.ds(dtidx * NWIN_PAD + 80, 16)] = jnp.where(
                    lane < 1, d80, _INF)

        tn = jnp.minimum(t + 1, T - 1)
        emit_cb(0)
        emit_pass(frame_b, 0)
        emit_cb(1)
        cp_c = pltpu.make_async_copy(vp_hbm.at[tn], frame_b, sem_b)
        cp_c.start()
        emit_pass(frame_a, 1)
        cp_c.wait()
        emit_cb(2)
        emit_pass(frame_b, 2)

        cp_a = pltpu.make_async_copy(vp_hbm.at[tn], frame_a, sem_a)
        cp_b = pltpu.make_async_copy(vp_hbm.at[t], frame_b, sem_b)
        cp_a.start()
        cp_b.start()

        @plsc.parallel_loop(0, NH, carry=total)
        def topk_body(q, tot):
            vs = [jnp.sort(dist[q, pl.ds(i * 16, 16)]) for i in range(NVREG)]
            while len(vs) > 1:
                nxt = []
                for i in range(0, len(vs) - 1, 2):
                    nxt.append(jnp.sort(jnp.minimum(vs[i], lax.rev(vs[i + 1], (0,)))))
                if len(vs) % 2:
                    nxt.append(vs[-1])
                vs = nxt
            qsum = jnp.sum(jnp.where(lane < K, vs[0], jnp.float32(0.0)))
            return tot + qsum

        total = topk_body
        cp_a.wait()
        cp_b.wait()
        return total

    total = lax.fori_loop(0, T, t_body, jnp.float32(0.0))
    stage[...] = jnp.where(lane < 1, total, jnp.float32(0.0))
    pltpu.sync_copy(stage, out_hbm.at[tid])


@jax.jit
def kernel(noisy, deno, fflow, bflow):
    del deno
    vid = noisy[0]
    p = PS // 2
    vp = jnp.pad(vid, ((0, 0), (0, 0), (p, p), (p, p)), mode="edge")
    vp = jnp.pad(vp, ((0, 0), (0, 0), (0, 0), (0, RSTR - ROWS)))
    vp_flat = jnp.pad(vp.reshape(T, FRAME), ((0, 0), (0, FRAME_PAD - FRAME)))
    flw = jnp.stack(
        [bflow[0], jnp.zeros_like(bflow[0]), fflow[0]], axis=1
    )
    flw_flat = flw.reshape(T * NDT * 2, H, W)

    mesh = plsc.VectorSubcoreMesh(core_axis_name="c", subcore_axis_name="s")
    run = functools.partial(
        pl.kernel,
        mesh=mesh,
        compiler_params=pltpu.CompilerParams(needs_layout_passes=False),
        out_type=jax.ShapeDtypeStruct((NTILES, 16), jnp.float32),
        scratch_types=[
            pltpu.VMEM((FRAME_PAD,), jnp.float32),
            pltpu.VMEM((FRAME_PAD,), jnp.float32),
            pltpu.VMEM((NH * NWIN_PAD,), jnp.int32),
            pltpu.VMEM((T * NDT * 2, W), jnp.float32),
            pltpu.VMEM((160,), jnp.int32),
            pltpu.VMEM((NH, NSLOT), jnp.float32),
            pltpu.VMEM((16,), jnp.float32),
            pltpu.SemaphoreType.DMA,
            pltpu.SemaphoreType.DMA,
        ],
    )(_sc_body)
    partials = run(vp_flat, flw_flat, jnp.asarray(_OFFS))
    return jnp.sum(partials) / jnp.float32(T * NH * NH * K)


# speedup vs baseline: 2.6123x; 2.6123x over previous
import functools

import jax
import jax.numpy as jnp
import numpy as np
from jax import lax
from jax.experimental import pallas as pl
from jax.experimental.pallas import tpu as pltpu
from jax.experimental.pallas import tpu_sc as plsc

WS = 9
WT = 1
PS = 7
K = 10
STRIDE0 = 4
T, C, H, W = 5, 3, 128, 128
ROWS = H + PS - 1
RSTR = 136
PLANE = ROWS * RSTR
FRAME = C * PLANE
FRAME_PAD = 54680
NH = H // STRIDE0
NWIN = WS * WS
NWIN_PAD = 96
NDT = 2 * WT + 1
NCHUNK = NWIN_PAD // 16
NSLOT = NDT * NWIN_PAD
NVREG = NSLOT // 16
NTILES = 32
NELEM = C * PS * PS
GSLICE = 17408

_INF = float(np.inf)
_MAGIC = float(1.5 * 2.0**23)

_OFFS = np.zeros((160,), np.int32)
_OFFS[:NELEM] = np.add.outer(
    np.arange(C) * PLANE,
    np.add.outer(np.arange(PS) * RSTR, np.arange(PS)).reshape(-1),
).reshape(-1)


def _sc_body(vp_hbm, flw_hbm, off_hbm, out_hbm,
             frame_a, frame_b, cb_v, flow_v, off_v, dist, stage,
             sem_a, sem_b):
    tid = lax.axis_index("s") * 2 + lax.axis_index("c")
    lane = lax.broadcasted_iota(jnp.int32, (16,), 0)
    qh_f = (tid * STRIDE0).astype(jnp.float32)
    pltpu.sync_copy(off_hbm, off_v)
    pltpu.sync_copy(flw_hbm.at[:, STRIDE0 * tid], flow_v)
    pltpu.sync_copy(vp_hbm.at[0], frame_a)
    pltpu.sync_copy(vp_hbm.at[0], frame_b)

    def t_body(t, total):
        def emit_cb(dtidx):
            frow = (t * NDT + dtidx) * 2
            zero16 = jnp.zeros((16,), jnp.int32)
            for g in range(2):
                qx = lane * STRIDE0 + 16 * STRIDE0 * g
                fw = plsc.load_gather(flow_v, [zero16 + frow, qx])
                fh = plsc.load_gather(flow_v, [zero16 + (frow + 1), qx])
                rw = (qx.astype(jnp.float32) + fw + _MAGIC) - _MAGIC
                rh = (qh_f + fh + _MAGIC) - _MAGIC
                qi = (lane + 16 * g) * NWIN_PAD

                @plsc.parallel_loop(0, NWIN)
                def w_body(w, rw=rw, rh=rh, qi=qi):
                    wi = (w // WS - WS // 2).astype(jnp.float32)
                    wj = (w % WS - WS // 2).astype(jnp.float32)
                    ch = jnp.minimum(jnp.maximum(rh + wi, 0.0), float(H - 1))
                    cw = jnp.minimum(jnp.maximum(rw + wj, 0.0), float(W - 1))
                    base = (ch * float(RSTR) + cw).astype(jnp.int32)
                    plsc.store_scatter(cb_v, [qi + w], base)

        def emit_pass(gref, dtidx):
            offv = [off_v[pl.ds(m * 16, 16)] for m in range(10)]

            @plsc.parallel_loop(0, NH)
            def q_body(q):
                qbase = tid * (STRIDE0 * RSTR) + q * STRIDE0
                bases = [cb_v[pl.ds(q * NWIN_PAD + k * 16, 16)]
                         for k in range(5)]
                accs = [jnp.zeros((16,), jnp.float32) for _ in range(5)]
                for c in range(C):
                    for i in range(PS):
                        off_ci = c * PLANE + i * RSTR
                        qrow = frame_a[pl.ds(qbase + off_ci, 16)]
                        for j in range(PS):
                            qv = qrow[j]
                            for k in range(5):
                                cv = plsc.load_gather(
                                    gref, [bases[k] + (off_ci + j)])
                                dd = cv - qv
                                accs[k] = accs[k] + dd * dd
                for k in range(5):
                    dist[q, pl.ds(dtidx * NWIN_PAD + k * 16, 16)] = accs[k]
                base80 = cb_v[pl.ds(q * NWIN_PAD + 80, 16)][0]
                acc80 = jnp.zeros((16,), jnp.float32)
                for m in range(10):
                    qe = plsc.load_gather(frame_a, [offv[m] + qbase])
                    ce = plsc.load_gather(gref, [offv[m] + base80])
                    dd = qe - ce
                    sq = dd * dd
                    if (m + 1) * 16 > NELEM:
                        sq = jnp.where(lane < NELEM - m * 16, sq,
                                       jnp.float32(0.0))
                    acc80 = acc80 + sq
                d80 = jnp.sum(acc80)
                dist[q, pl.ds(dtidx * NWIN_PAD + 80, 16)] = jnp.where(
                    lane < 1, d80, _INF)

        tn = jnp.minimum(t + 1, T - 1)
        emit_cb(0)
        emit_pass(frame_b, 0)
        emit_cb(1)
        cp_c = pltpu.make_async_copy(vp_hbm.at[tn], frame_b, sem_b)
        cp_c.start()
        emit_pass(frame_a, 1)
        cp_c.wait()
        emit_cb(2)
        emit_pass(frame_b, 2)

        cp_a = pltpu.make_async_copy(vp_hbm.at[tn], frame_a, sem_a)
        cp_b = pltpu.make_async_copy(vp_hbm.at[t], frame_b, sem_b)
        cp_a.start()
        cp_b.start()

        @plsc.parallel_loop(0, NH, carry=total)
        def topk_body(q, tot):
            vs = [jnp.sort(dist[q, pl.ds(i * 16, 16)]) for i in range(NVREG)]
            while len(vs) > 1:
                nxt = []
                for i in range(0, len(vs) - 1, 2):
                    nxt.append(jnp.sort(jnp.minimum(vs[i], lax.rev(vs[i + 1], (0,)))))
                if len(vs) % 2:
                    nxt.append(vs[-1])
                vs = nxt
            qsum = jnp.sum(jnp.where(lane < K, vs[0], jnp.float32(0.0)))
            return tot + qsum

        total = topk_body
        cp_a.wait()
        cp_b.wait()
        return total

    total = lax.fori_loop(0, T, t_body, jnp.float32(0.0))
    stage[...] = jnp.where(lane < 1, total, jnp.float32(0.0))
    pltpu.sync_copy(stage, out_hbm.at[tid])


@jax.jit
def kernel(noisy, deno, fflow, bflow):
    del deno
    vid = noisy[0]
    p = PS // 2
    vp = jnp.pad(vid, ((0, 0), (0, 0), (p, p), (p, p)), mode="edge")
    vp = jnp.pad(vp, ((0, 0), (0, 0), (0, 0), (0, RSTR - ROWS)))
    vp_flat = jnp.pad(vp.reshape(T, FRAME), ((0, 0), (0, FRAME_PAD - FRAME)))
    flw = jnp.stack(
        [bflow[0], jnp.zeros_like(bflow[0]), fflow[0]], axis=1
    )
    flw_flat = flw.reshape(T * NDT * 2, H, W)

    mesh = plsc.VectorSubcoreMesh(core_axis_name="c", subcore_axis_name="s")
    run = functools.partial(
        pl.kernel,
        mesh=mesh,
        compiler_params=pltpu.CompilerParams(needs_layout_passes=False),
        out_type=jax.ShapeDtypeStruct((NTILES, 16), jnp.float32),
        scratch_types=[
            pltpu.VMEM((FRAME_PAD,), jnp.float32),
            pltpu.VMEM((FRAME_PAD,), jnp.float32),
            pltpu.VMEM((NH * NWIN_PAD,), jnp.int32),
            pltpu.VMEM((T * NDT * 2, W), jnp.float32),
            pltpu.VMEM((160,), jnp.int32),
            pltpu.VMEM((NH, NSLOT), jnp.float32),
            pltpu.VMEM((16,), jnp.float32),
            pltpu.SemaphoreType.DMA,
            pltpu.SemaphoreType.DMA,
        ],
    )(_sc_body)
    partials = run(vp_flat, flw_flat, jnp.asarray(_OFFS))
    return jnp.sum(partials) / jnp.float32(T * NH * NH * K)
